# baseline (speedup 1.0000x reference)
import functools

import jax
import jax.numpy as jnp
from jax import lax
from jax.experimental import pallas as pl
from jax.experimental.pallas import tpu as pltpu
from jax.experimental.pallas import tpu_sc as plsc

DIN = 1024
DOUT = 1024
HD = DIN // 2
E = 8
EPS = 1e-6
T = 4096
A = 2 * T
B2 = 512
NBLK = (A + E * (B2 - 1) + B2 - 1) // B2
PAD = NBLK * B2
NW = 32
CPW = T // NW
SZ = 32
CSZ = 16
CHUNK = 512


def _pack(lo_f32, hi_f32):
    lo = lax.bitcast_convert_type(lo_f32.astype(jnp.bfloat16), jnp.uint16)
    hi = lax.bitcast_convert_type(hi_f32.astype(jnp.bfloat16), jnp.uint16)
    word = lo.astype(jnp.uint32) | (hi.astype(jnp.uint32) << 16)
    return lax.bitcast_convert_type(word, jnp.int32)


def _unpack_bf16(words_i32):
    w = lax.bitcast_convert_type(words_i32, jnp.uint32)
    lo = lax.bitcast_convert_type((w & 0xFFFF).astype(jnp.uint16), jnp.bfloat16)
    hi = lax.bitcast_convert_type((w >> 16).astype(jnp.uint16), jnp.bfloat16)
    return lo, hi


def _router_body(x_ref, gw_ref, gb_ref, gamma_ref, beta_ref,
                 xp_ref, pw_ref, d0_ref, d1_ref, be_ref):
    logits = lax.dot_general(
        x_ref[...], gw_ref[...], (((1,), (1,)), ((), ())),
        preferred_element_type=jnp.float32) + gb_ref[...]
    gx = jnp.sqrt(jnp.sum(logits * logits, axis=1, keepdims=True))
    nx = gx / (jnp.mean(gx, axis=0, keepdims=True) + EPS)
    logits = gamma_ref[...] * (logits * nx) + beta_ref[...]
    m = jnp.max(logits, axis=1, keepdims=True)
    p = jnp.exp(logits - m)
    p = p / jnp.sum(p, axis=1, keepdims=True)
    ii = lax.broadcasted_iota(jnp.int32, p.shape, 1)
    m1 = jnp.max(p, axis=1, keepdims=True)
    i1 = jnp.min(jnp.where(p == m1, ii, E), axis=1, keepdims=True)
    s1 = ii == i1
    pm = jnp.where(s1, -jnp.inf, p)
    m2 = jnp.max(pm, axis=1, keepdims=True)
    i2 = jnp.min(jnp.where(pm == m2, ii, E), axis=1, keepdims=True)
    s2 = ii == i2
    h0 = s1.astype(jnp.float32)
    h1 = s2.astype(jnp.float32)
    ir = lax.broadcasted_iota(jnp.int32, (CHUNK, CHUNK), 0)
    ic = lax.broadcasted_iota(jnp.int32, (CHUNK, CHUNK), 1)
    tri = (ic < ir).astype(jnp.float32)
    carry = jnp.zeros((1, E), jnp.float32)
    ranks = []
    for h in (h0, h1):
        chunks = []
        for s in range(0, T, CHUNK):
            hc = lax.slice(h, (s, 0), (s + CHUNK, E))
            r = lax.dot_general(tri, hc, (((1,), (0,)), ((), ())),
                                preferred_element_type=jnp.float32) + carry
            carry = carry + jnp.sum(hc, axis=0, keepdims=True)
            chunks.append(r)
        ranks.append(jnp.concatenate(chunks, axis=0))
    rank0, rank1 = ranks
    cnt = carry
    padcnt = jnp.ceil(cnt / B2) * B2
    er = lax.broadcasted_iota(jnp.int32, (E, E), 0)
    ec = lax.broadcasted_iota(jnp.int32, (E, E), 1)
    mtri = (er < ec).astype(jnp.float32)
    off = lax.dot_general(padcnt, mtri, (((1,), (0,)), ((), ())),
                          preferred_element_type=jnp.float32)
    d0_ref[...] = jnp.sum((off + rank0) * h0, axis=1,
                          keepdims=True).astype(jnp.int32)
    d1_ref[...] = jnp.sum((off + rank1) * h1, axis=1,
                          keepdims=True).astype(jnp.int32)
    xp_ref[...] = _pack(lax.slice(x_ref[...], (0, 0), (T, HD)),
                        lax.slice(x_ref[...], (0, HD), (T, DIN)))
    pw_ref[...] = jnp.broadcast_to(jnp.concatenate([m1, m2], axis=0), (A, 128))
    ends = (off + padcnt).astype(jnp.int32)
    bi = lax.broadcasted_iota(jnp.int32, (32, E), 0) * B2
    becnt = jnp.sum((bi >= ends).astype(jnp.int32), axis=1, keepdims=True)
    total = jnp.sum(padcnt).astype(jnp.int32)
    bvalid = (lax.broadcasted_iota(jnp.int32, (32, 1), 0) * B2) < total
    be_ref[...] = jnp.where(bvalid, becnt, -1)


def _mm_body(be_ref, xs_ref, sw_ref, w_ref, b_ref, ys_ref):
    b = pl.program_id(0)
    e = be_ref[b]

    @pl.when(e >= 0)
    def _():
        lo, hi = _unpack_bf16(xs_ref[...])
        xb = jnp.concatenate([lo, hi], axis=1)
        wb = w_ref[0].astype(jnp.bfloat16)
        y = lax.dot_general(xb, wb, (((1,), (1,)), ((), ())),
                            preferred_element_type=jnp.float32)
        y = (y + b_ref[0]) * sw_ref[:, 0:1]
        ys_ref[...] = _pack(lax.slice(y, (0, 0), (B2, HD)),
                            lax.slice(y, (0, HD), (B2, DOUT)))

    @pl.when(e < 0)
    def _():
        ys_ref[...] = jnp.zeros_like(ys_ref)


@functools.lru_cache(maxsize=None)
def _sc_kernels():
    mesh = plsc.VectorSubcoreMesh(core_axis_name="c", subcore_axis_name="s")
    nj = CPW // SZ

    @functools.partial(
        pl.kernel, mesh=mesh,
        out_type=[jax.ShapeDtypeStruct((PAD, HD), jnp.int32),
                  jax.ShapeDtypeStruct((PAD, 128), jnp.float32)],
        scratch_types=[pltpu.VMEM((nj, SZ), jnp.int32),
                       pltpu.VMEM((nj, SZ), jnp.int32),
                       pltpu.VMEM((SZ, HD), jnp.int32),
                       pltpu.VMEM((SZ, HD), jnp.int32),
                       pltpu.VMEM((SZ, 128), jnp.float32),
                       pltpu.VMEM((SZ, 128), jnp.float32),
                       pltpu.VMEM((SZ, 128), jnp.float32),
                       pltpu.VMEM((SZ, 128), jnp.float32),
                       pltpu.SemaphoreType.DMA,
                       pltpu.SemaphoreType.DMA])
    def dispatch(xp_hbm, pw_hbm, d0_hbm, d1_hbm, xs_hbm, sw_hbm,
                 dva, dvb, xb0, xb1, pa0, pa1, pb0, pb1, lsem, ssem):
        w = lax.axis_index("s") * 2 + lax.axis_index("c")
        tb = w * CPW
        for j in range(nj):
            pltpu.sync_copy(d0_hbm.at[pl.ds(tb + j * SZ, SZ)], dva.at[j])
            pltpu.sync_copy(d1_hbm.at[pl.ds(tb + j * SZ, SZ)], dvb.at[j])
        xbufs = (xb0, xb1)
        pabufs = (pa0, pa1)
        pbbufs = (pb0, pb1)

        def fire_loads(j, b):
            o = pl.ds(tb + j * SZ, SZ)
            return (pltpu.async_copy(xp_hbm.at[o], xbufs[b], lsem),
                    pltpu.async_copy(pw_hbm.at[o], pabufs[b], lsem),
                    pltpu.async_copy(pw_hbm.at[pl.ds(T + tb + j * SZ, SZ)],
                                     pbbufs[b], lsem))

        def fire_scats(j, b):
            return (pltpu.async_copy(xbufs[b], xs_hbm.at[dva.at[j]], ssem),
                    pltpu.async_copy(xbufs[b], xs_hbm.at[dvb.at[j]], ssem),
                    pltpu.async_copy(pabufs[b], sw_hbm.at[dva.at[j]], ssem),
                    pltpu.async_copy(pbbufs[b], sw_hbm.at[dvb.at[j]], ssem))

        loads = fire_loads(0, 0)
        scat_prev = None
        for j in range(nj):
            b = j % 2
            for h in loads:
                h.wait()
            scats = fire_scats(j, b)
            if scat_prev is not None:
                for h in scat_prev:
                    h.wait()
            if j + 1 < nj:
                loads = fire_loads(j + 1, (j + 1) % 2)
            scat_prev = scats
        for h in scat_prev:
            h.wait()

    nc = CPW // CSZ

    @functools.partial(
        pl.kernel, mesh=mesh,
        out_type=jax.ShapeDtypeStruct((T, DOUT), jnp.float32),
        scratch_types=[pltpu.VMEM((nc, CSZ), jnp.int32),
                       pltpu.VMEM((nc, CSZ), jnp.int32),
                       pltpu.VMEM((CSZ, HD), jnp.int32),
                       pltpu.VMEM((CSZ, HD), jnp.int32),
                       pltpu.VMEM((CSZ, HD), jnp.int32),
                       pltpu.VMEM((CSZ, HD), jnp.int32),
                       pltpu.VMEM((CSZ, DOUT), jnp.float32),
                       pltpu.SemaphoreType.DMA])
    def combine(ys_hbm, d0_hbm, d1_hbm, out_hbm, dv0, dv1,
                r0a, r1a, r0b, r1b, ob, gsem):
        w = lax.axis_index("s") * 2 + lax.axis_index("c")
        tb = w * CPW
        for j in range(nc):
            pltpu.sync_copy(d0_hbm.at[pl.ds(tb + j * CSZ, CSZ)], dv0.at[j])
            pltpu.sync_copy(d1_hbm.at[pl.ds(tb + j * CSZ, CSZ)], dv1.at[j])
        bufs = ((r0a, r1a), (r0b, r1b))
        pend = [None, None]
        pend[0] = (pltpu.async_copy(ys_hbm.at[dv0.at[0]], r0a, gsem),
                   pltpu.async_copy(ys_hbm.at[dv1.at[0]], r1a, gsem))
        for j in range(nc):
            c0, c1 = bufs[j % 2]
            g0, g1 = pend[j % 2]
            g0.wait()
            g1.wait()
            if j + 1 < nc:
                nb0, nb1 = bufs[(j + 1) % 2]
                pend[(j + 1) % 2] = (
                    pltpu.async_copy(ys_hbm.at[dv0.at[j + 1]], nb0, gsem),
                    pltpu.async_copy(ys_hbm.at[dv1.at[j + 1]], nb1, gsem))

            def body(i, _):
                r = i >> 5
                c = (i & 31) * 16
                v0 = c0[r, pl.ds(c, 16)]
                v1 = c1[r, pl.ds(c, 16)]
                hmask = jnp.int32(-65536)
                a0 = lax.bitcast_convert_type(v0 << 16, jnp.float32)
                b0 = lax.bitcast_convert_type(v0 & hmask, jnp.float32)
                a1 = lax.bitcast_convert_type(v1 << 16, jnp.float32)
                b1 = lax.bitcast_convert_type(v1 & hmask, jnp.float32)
                ob[r, pl.ds(c, 16)] = a0 + a1
                ob[r, pl.ds(HD + c, 16)] = b0 + b1
                return 0

            lax.fori_loop(0, CSZ * (HD // 16), body, 0, unroll=8)
            pltpu.sync_copy(ob, out_hbm.at[pl.ds(tb + j * CSZ, CSZ)])

    return dispatch, combine


@jax.jit
def kernel(x, gate_W, gate_b, expert_W, expert_b, gamma, beta):
    xp, pw, d0, d1, be = pl.pallas_call(
        _router_body,
        out_shape=[
            jax.ShapeDtypeStruct((T, HD), jnp.int32),
            jax.ShapeDtypeStruct((A, 128), jnp.float32),
            jax.ShapeDtypeStruct((T, 1), jnp.int32),
            jax.ShapeDtypeStruct((T, 1), jnp.int32),
            jax.ShapeDtypeStruct((32, 1), jnp.int32),
        ],
    )(x, gate_W, gate_b.reshape(1, E), gamma, beta)

    d0f = d0.reshape(T)
    d1f = d1.reshape(T)

    dispatch, combine = _sc_kernels()
    xs, sw = dispatch(xp, pw, d0f, d1f)

    wt = expert_W.reshape(E, DOUT, DIN)
    b2d = expert_b.reshape(E, 1, DOUT)
    grid_spec = pltpu.PrefetchScalarGridSpec(
        num_scalar_prefetch=1,
        grid=(NBLK,),
        in_specs=[
            pl.BlockSpec((B2, HD),
                         lambda b, be_r: (jnp.where(be_r[b] >= 0, b, 0), 0)),
            pl.BlockSpec((B2, 128),
                         lambda b, be_r: (jnp.where(be_r[b] >= 0, b, 0), 0)),
            pl.BlockSpec((1, DOUT, DIN),
                         lambda b, be_r: (jnp.maximum(be_r[b], 0), 0, 0)),
            pl.BlockSpec((1, 1, DOUT),
                         lambda b, be_r: (jnp.maximum(be_r[b], 0), 0, 0)),
        ],
        out_specs=pl.BlockSpec((B2, HD), lambda b, be_r: (b, 0)),
    )
    ys = pl.pallas_call(
        _mm_body,
        grid_spec=grid_spec,
        out_shape=jax.ShapeDtypeStruct((PAD, HD), jnp.int32),
    )(be.reshape(32), xs, sw, wt, b2d)

    return combine(ys, d0f, d1f)


# speedup vs baseline: 1.0315x; 1.0315x over previous
import functools

import jax
import jax.numpy as jnp
from jax import lax
from jax.experimental import pallas as pl
from jax.experimental.pallas import tpu as pltpu
from jax.experimental.pallas import tpu_sc as plsc

DIN = 1024
DOUT = 1024
HD = DIN // 2
E = 8
EPS = 1e-6
T = 4096
A = 2 * T
B2 = 512
NBLK = (A + E * (B2 - 1) + B2 - 1) // B2
PAD = NBLK * B2
NW = 32
CPW = T // NW
SZ = 64
CSZ = 32
CHUNK = 512


def _pack(lo_f32, hi_f32):
    lo = lax.bitcast_convert_type(lo_f32.astype(jnp.bfloat16), jnp.uint16)
    hi = lax.bitcast_convert_type(hi_f32.astype(jnp.bfloat16), jnp.uint16)
    word = lo.astype(jnp.uint32) | (hi.astype(jnp.uint32) << 16)
    return lax.bitcast_convert_type(word, jnp.int32)


_pack_bf = _pack


def _unpack_bf16(words_i32):
    w = lax.bitcast_convert_type(words_i32, jnp.uint32)
    lo = lax.bitcast_convert_type((w & 0xFFFF).astype(jnp.uint16), jnp.bfloat16)
    hi = lax.bitcast_convert_type((w >> 16).astype(jnp.uint16), jnp.bfloat16)
    return lo, hi


def _router_body(x_ref, gw_ref, gb_ref, gamma_ref, beta_ref,
                 xp_ref, pw_ref, d0_ref, d1_ref, be_ref):
    logits = lax.dot_general(
        x_ref[...], gw_ref[...], (((1,), (1,)), ((), ())),
        preferred_element_type=jnp.float32) + gb_ref[...]
    gx = jnp.sqrt(jnp.sum(logits * logits, axis=1, keepdims=True))
    nx = gx / (jnp.mean(gx, axis=0, keepdims=True) + EPS)
    logits = gamma_ref[...] * (logits * nx) + beta_ref[...]
    m = jnp.max(logits, axis=1, keepdims=True)
    p = jnp.exp(logits - m)
    p = p / jnp.sum(p, axis=1, keepdims=True)
    ii = lax.broadcasted_iota(jnp.int32, p.shape, 1)
    m1 = jnp.max(p, axis=1, keepdims=True)
    i1 = jnp.min(jnp.where(p == m1, ii, E), axis=1, keepdims=True)
    s1 = ii == i1
    pm = jnp.where(s1, -jnp.inf, p)
    m2 = jnp.max(pm, axis=1, keepdims=True)
    i2 = jnp.min(jnp.where(pm == m2, ii, E), axis=1, keepdims=True)
    s2 = ii == i2
    h0 = s1.astype(jnp.float32)
    h1 = s2.astype(jnp.float32)
    ir = lax.broadcasted_iota(jnp.int32, (CHUNK, CHUNK), 0)
    ic = lax.broadcasted_iota(jnp.int32, (CHUNK, CHUNK), 1)
    tri = (ic < ir).astype(jnp.float32)
    carry = jnp.zeros((1, E), jnp.float32)
    ranks = []
    for h in (h0, h1):
        chunks = []
        for s in range(0, T, CHUNK):
            hc = lax.slice(h, (s, 0), (s + CHUNK, E))
            r = lax.dot_general(tri, hc, (((1,), (0,)), ((), ())),
                                preferred_element_type=jnp.float32) + carry
            carry = carry + jnp.sum(hc, axis=0, keepdims=True)
            chunks.append(r)
        ranks.append(jnp.concatenate(chunks, axis=0))
    rank0, rank1 = ranks
    cnt = carry
    padcnt = jnp.ceil(cnt / B2) * B2
    er = lax.broadcasted_iota(jnp.int32, (E, E), 0)
    ec = lax.broadcasted_iota(jnp.int32, (E, E), 1)
    mtri = (er < ec).astype(jnp.float32)
    off = lax.dot_general(padcnt, mtri, (((1,), (0,)), ((), ())),
                          preferred_element_type=jnp.float32)
    d0_ref[...] = jnp.sum((off + rank0) * h0, axis=1,
                          keepdims=True).astype(jnp.int32)
    d1_ref[...] = jnp.sum((off + rank1) * h1, axis=1,
                          keepdims=True).astype(jnp.int32)
    xp_ref[...] = _pack(lax.slice(x_ref[...], (0, 0), (T, HD)),
                        lax.slice(x_ref[...], (0, HD), (T, DIN)))
    pw_ref[...] = jnp.broadcast_to(jnp.concatenate([m1, m2], axis=0), (A, 128))
    ends = (off + padcnt).astype(jnp.int32)
    bi = lax.broadcasted_iota(jnp.int32, (32, E), 0) * B2
    becnt = jnp.sum((bi >= ends).astype(jnp.int32), axis=1, keepdims=True)
    total = jnp.sum(padcnt).astype(jnp.int32)
    bvalid = (lax.broadcasted_iota(jnp.int32, (32, 1), 0) * B2) < total
    be_ref[...] = jnp.where(bvalid, becnt, -1)


def _mm_body(be_ref, xs_ref, sw_ref, w00_ref, w01_ref, w10_ref, w11_ref,
             bl_ref, bh_ref, ys_ref):
    e = be_ref[pl.program_id(0)]

    @pl.when(e >= 0)
    def _():
        lo, hi = _unpack_bf16(xs_ref[...])
        s = sw_ref[:, 0:1]

        def half(wl_ref, wh_ref, bias_ref):
            wl = wl_ref[0].astype(jnp.bfloat16)
            wh = wh_ref[0].astype(jnp.bfloat16)
            y = lax.dot_general(lo, wl, (((1,), (1,)), ((), ())),
                                preferred_element_type=jnp.float32)
            y = y + lax.dot_general(hi, wh, (((1,), (1,)), ((), ())),
                                    preferred_element_type=jnp.float32)
            return (y + bias_ref[0]) * s

        ylo = half(w00_ref, w01_ref, bl_ref)
        yhi = half(w10_ref, w11_ref, bh_ref)
        ys_ref[...] = _pack_bf(ylo, yhi)

    @pl.when(e < 0)
    def _():
        ys_ref[...] = jnp.zeros_like(ys_ref)


@functools.lru_cache(maxsize=None)
def _sc_kernels():
    mesh = plsc.VectorSubcoreMesh(core_axis_name="c", subcore_axis_name="s")
    nj = CPW // SZ

    @functools.partial(
        pl.kernel, mesh=mesh,
        out_type=[jax.ShapeDtypeStruct((PAD, HD), jnp.int32),
                  jax.ShapeDtypeStruct((PAD, 128), jnp.float32)],
        scratch_types=[pltpu.VMEM((nj, SZ), jnp.int32),
                       pltpu.VMEM((nj, SZ), jnp.int32),
                       pltpu.VMEM((SZ, HD), jnp.int32),
                       pltpu.VMEM((SZ, HD), jnp.int32),
                       pltpu.VMEM((SZ, 128), jnp.float32),
                       pltpu.VMEM((SZ, 128), jnp.float32),
                       pltpu.VMEM((SZ, 128), jnp.float32),
                       pltpu.VMEM((SZ, 128), jnp.float32),
                       pltpu.SemaphoreType.DMA,
                       pltpu.SemaphoreType.DMA])
    def dispatch(xp_hbm, pw_hbm, d0_hbm, d1_hbm, xs_hbm, sw_hbm,
                 dva, dvb, xb0, xb1, pa0, pa1, pb0, pb1, lsem, ssem):
        w = lax.axis_index("s") * 2 + lax.axis_index("c")
        tb = w * CPW
        for j in range(nj):
            pltpu.sync_copy(d0_hbm.at[pl.ds(tb + j * SZ, SZ)], dva.at[j])
            pltpu.sync_copy(d1_hbm.at[pl.ds(tb + j * SZ, SZ)], dvb.at[j])
        xbufs = (xb0, xb1)
        pabufs = (pa0, pa1)
        pbbufs = (pb0, pb1)

        def fire_loads(j, b):
            o = pl.ds(tb + j * SZ, SZ)
            return (pltpu.async_copy(xp_hbm.at[o], xbufs[b], lsem),
                    pltpu.async_copy(pw_hbm.at[o], pabufs[b], lsem),
                    pltpu.async_copy(pw_hbm.at[pl.ds(T + tb + j * SZ, SZ)],
                                     pbbufs[b], lsem))

        def fire_scats(j, b):
            return (pltpu.async_copy(xbufs[b], xs_hbm.at[dva.at[j]], ssem),
                    pltpu.async_copy(xbufs[b], xs_hbm.at[dvb.at[j]], ssem),
                    pltpu.async_copy(pabufs[b], sw_hbm.at[dva.at[j]], ssem),
                    pltpu.async_copy(pbbufs[b], sw_hbm.at[dvb.at[j]], ssem))

        loads = fire_loads(0, 0)
        scat_prev = None
        for j in range(nj):
            b = j % 2
            for h in loads:
                h.wait()
            scats = fire_scats(j, b)
            if scat_prev is not None:
                for h in scat_prev:
                    h.wait()
            if j + 1 < nj:
                loads = fire_loads(j + 1, (j + 1) % 2)
            scat_prev = scats
        for h in scat_prev:
            h.wait()

    nc = CPW // CSZ

    @functools.partial(
        pl.kernel, mesh=mesh,
        out_type=jax.ShapeDtypeStruct((T, DOUT), jnp.float32),
        scratch_types=[pltpu.VMEM((nc, CSZ), jnp.int32),
                       pltpu.VMEM((nc, CSZ), jnp.int32),
                       pltpu.VMEM((CSZ, HD), jnp.int32),
                       pltpu.VMEM((CSZ, HD), jnp.int32),
                       pltpu.VMEM((CSZ, HD), jnp.int32),
                       pltpu.VMEM((CSZ, HD), jnp.int32),
                       pltpu.VMEM((CSZ, DOUT), jnp.float32),
                       pltpu.SemaphoreType.DMA])
    def combine(ys_hbm, d0_hbm, d1_hbm, out_hbm, dv0, dv1,
                r0a, r1a, r0b, r1b, ob, gsem):
        w = lax.axis_index("s") * 2 + lax.axis_index("c")
        tb = w * CPW
        for j in range(nc):
            pltpu.sync_copy(d0_hbm.at[pl.ds(tb + j * CSZ, CSZ)], dv0.at[j])
            pltpu.sync_copy(d1_hbm.at[pl.ds(tb + j * CSZ, CSZ)], dv1.at[j])
        bufs = ((r0a, r1a), (r0b, r1b))
        pend = [None, None]
        pend[0] = (pltpu.async_copy(ys_hbm.at[dv0.at[0]], r0a, gsem),
                   pltpu.async_copy(ys_hbm.at[dv1.at[0]], r1a, gsem))
        for j in range(nc):
            c0, c1 = bufs[j % 2]
            g0, g1 = pend[j % 2]
            g0.wait()
            g1.wait()
            if j + 1 < nc:
                nb0, nb1 = bufs[(j + 1) % 2]
                pend[(j + 1) % 2] = (
                    pltpu.async_copy(ys_hbm.at[dv0.at[j + 1]], nb0, gsem),
                    pltpu.async_copy(ys_hbm.at[dv1.at[j + 1]], nb1, gsem))

            def body(r, _):
                hmask = jnp.int32(-65536)
                for cc in range(HD // 16):
                    c = cc * 16
                    v0 = c0[r, pl.ds(c, 16)]
                    v1 = c1[r, pl.ds(c, 16)]
                    a0 = lax.bitcast_convert_type(v0 << 16, jnp.float32)
                    b0 = lax.bitcast_convert_type(v0 & hmask, jnp.float32)
                    a1 = lax.bitcast_convert_type(v1 << 16, jnp.float32)
                    b1 = lax.bitcast_convert_type(v1 & hmask, jnp.float32)
                    ob[r, pl.ds(c, 16)] = a0 + a1
                    ob[r, pl.ds(HD + c, 16)] = b0 + b1
                return 0

            lax.fori_loop(0, CSZ, body, 0)
            pltpu.sync_copy(ob, out_hbm.at[pl.ds(tb + j * CSZ, CSZ)])

    return dispatch, combine


@jax.jit
def kernel(x, gate_W, gate_b, expert_W, expert_b, gamma, beta):
    xp, pw, d0, d1, be = pl.pallas_call(
        _router_body,
        out_shape=[
            jax.ShapeDtypeStruct((T, HD), jnp.int32),
            jax.ShapeDtypeStruct((A, 128), jnp.float32),
            jax.ShapeDtypeStruct((T, 1), jnp.int32),
            jax.ShapeDtypeStruct((T, 1), jnp.int32),
            jax.ShapeDtypeStruct((32, 1), jnp.int32),
        ],
    )(x, gate_W, gate_b.reshape(1, E), gamma, beta)

    d0f = d0.reshape(T)
    d1f = d1.reshape(T)

    dispatch, combine = _sc_kernels()
    xs, sw = dispatch(xp, pw, d0f, d1f)

    wt = expert_W.reshape(E, DOUT, DIN)
    b2d = expert_b.reshape(E, 1, DOUT)
    grid_spec = pltpu.PrefetchScalarGridSpec(
        num_scalar_prefetch=1,
        grid=(NBLK,),
        in_specs=[
            pl.BlockSpec((B2, HD),
                         lambda b, be_r: (jnp.where(be_r[b] >= 0, b, 0), 0)),
            pl.BlockSpec((B2, 128),
                         lambda b, be_r: (jnp.where(be_r[b] >= 0, b, 0), 0)),
            pl.BlockSpec((1, HD, HD),
                         lambda b, be_r: (jnp.maximum(be_r[b], 0), 0, 0)),
            pl.BlockSpec((1, HD, HD),
                         lambda b, be_r: (jnp.maximum(be_r[b], 0), 0, 1)),
            pl.BlockSpec((1, HD, HD),
                         lambda b, be_r: (jnp.maximum(be_r[b], 0), 1, 0)),
            pl.BlockSpec((1, HD, HD),
                         lambda b, be_r: (jnp.maximum(be_r[b], 0), 1, 1)),
            pl.BlockSpec((1, 1, HD),
                         lambda b, be_r: (jnp.maximum(be_r[b], 0), 0, 0)),
            pl.BlockSpec((1, 1, HD),
                         lambda b, be_r: (jnp.maximum(be_r[b], 0), 0, 1)),
        ],
        out_specs=pl.BlockSpec((B2, HD), lambda b, be_r: (b, 0)),
    )
    ys = pl.pallas_call(
        _mm_body,
        grid_spec=grid_spec,
        out_shape=jax.ShapeDtypeStruct((PAD, HD), jnp.int32),
    )(be.reshape(32), xs, sw, wt, wt, wt, wt, b2d, b2d)

    return combine(ys, d0f, d1f)
